# baseline (speedup 1.0000x reference)
import jax
import jax.numpy as jnp
from jax import lax
from jax.experimental import pallas as pl
from jax.experimental.pallas import tpu as pltpu
from jax.experimental.pallas import tpu_sc as plsc

N = 10000
E = 160000
F = 16

NC = 2
NS = 16
NW = NC * NS

EPT = E // NW
CE = 128
NFULL = EPT // CE
TAIL_E = EPT - NFULL * CE

CN = 128
NJ_FULL = N // CN
TAIL_N = N - NJ_FULL * CN

ROWS_A = 624
ROWS_LAST = N - (NS - 1) * ROWS_A

_GDN = lax.GatherDimensionNumbers(
    offset_dims=(), collapsed_slice_dims=(0,), start_index_map=(0,))


def _bcast_lane(x_row, i):
    idx = jnp.full((F, 1), i, dtype=jnp.int32)
    return lax.gather(x_row, idx, dimension_numbers=_GDN, slice_sizes=(1,),
                      mode=lax.GatherScatterMode.PROMISE_IN_BOUNDS)


def _compute_chunk(cnt, x_ref, w_ref, b_ref, m_ref):
    def body(e, _):
        x_row = x_ref[e, :]
        acc = b_ref[e, :]
        for i in range(F):
            acc = acc + _bcast_lane(x_row, i) * w_ref[e, i, :]
        m_ref[e, :] = acc
        return 0
    lax.fori_loop(0, cnt, body, 0)


def _gnn_body(feat, lw, w_hbm, mb, hb, src_h, dst_h, p_out, loop_out,
              w_buf, x_buf, b_buf, m_buf, sidx, didx, didx_t, shared):
    cid = lax.axis_index("c")
    sid = lax.axis_index("s")
    wid = cid * NS + sid

    row0 = pl.multiple_of(sid * ROWS_A, 8)

    def zbody(i, _):
        m_buf[i, :] = jnp.zeros((F,), jnp.float32)
        return 0
    lax.fori_loop(0, CE, zbody, 0)
    for k in range(4):
        pltpu.sync_copy(m_buf.at[pl.ds(0, CE)],
                        shared.at[pl.ds(pl.multiple_of(row0 + k * CE, 8), CE)])

    @pl.when(sid < NS - 1)
    def _zero_tail_a():
        pltpu.sync_copy(m_buf.at[pl.ds(0, ROWS_A - 4 * CE)],
                        shared.at[pl.ds(pl.multiple_of(row0 + 4 * CE, 8),
                                        ROWS_A - 4 * CE)])

    @pl.when(sid == NS - 1)
    def _zero_tail_b():
        pltpu.sync_copy(m_buf.at[pl.ds(0, CE)],
                        shared.at[pl.ds(pl.multiple_of(row0 + 4 * CE, 8), CE)])

    plsc.subcore_barrier()

    def loop_chunk(noff, cnt):
        pltpu.sync_copy(lw.at[pl.ds(noff, cnt)], w_buf.at[pl.ds(0, cnt)])
        pltpu.sync_copy(feat.at[pl.ds(noff, cnt)], x_buf.at[pl.ds(0, cnt)])
        pltpu.sync_copy(hb.at[pl.ds(noff, cnt)], b_buf.at[pl.ds(0, cnt)])
        _compute_chunk(cnt, x_buf, w_buf, b_buf, m_buf)
        pltpu.sync_copy(m_buf.at[pl.ds(0, cnt)], loop_out.at[pl.ds(noff, cnt)])

    @pl.when(cid == 0)
    def _loop_term():
        for k in range(5):
            j = sid + NS * k

            @pl.when(j < NJ_FULL)
            def _one():
                loop_chunk(pl.multiple_of(j * CN, 8), CN)

        @pl.when(sid == NS - 1)
        def _node_tail():
            loop_chunk(NJ_FULL * CN, TAIL_N)

    ebase = wid * EPT

    def edge_chunk(off, cnt, didx_ref):
        off = pl.multiple_of(off, 8)
        pltpu.sync_copy(src_h.at[pl.ds(off, cnt)], sidx.at[pl.ds(0, cnt)])
        pltpu.sync_copy(dst_h.at[pl.ds(off, cnt)], didx_ref)
        pltpu.sync_copy(w_hbm.at[pl.ds(off, cnt)], w_buf.at[pl.ds(0, cnt)])
        pltpu.sync_copy(mb.at[pl.ds(off, cnt)], b_buf.at[pl.ds(0, cnt)])
        pltpu.sync_copy(feat.at[sidx.at[pl.ds(0, cnt)]],
                        x_buf.at[pl.ds(0, cnt)])
        _compute_chunk(cnt, x_buf, w_buf, b_buf, m_buf)
        pltpu.sync_copy(m_buf.at[pl.ds(0, cnt)], shared.at[didx_ref],
                        add=True)

    def echunk_body(c, _):
        edge_chunk(ebase + c * CE, CE, didx)
        return 0
    lax.fori_loop(0, NFULL, echunk_body, 0)
    edge_chunk(ebase + NFULL * CE, TAIL_E, didx_t)

    plsc.subcore_barrier()

    @pl.when(sid < NS - 1)
    def _pub_a():
        pltpu.sync_copy(shared.at[pl.ds(row0, ROWS_A)],
                        p_out.at[cid, pl.ds(row0, ROWS_A)])

    @pl.when(sid == NS - 1)
    def _pub_b():
        pltpu.sync_copy(shared.at[pl.ds(row0, ROWS_LAST)],
                        p_out.at[cid, pl.ds(row0, ROWS_LAST)])


def _make_sc_call():
    mesh = plsc.VectorSubcoreMesh(core_axis_name="c", subcore_axis_name="s")
    return pl.kernel(
        _gnn_body,
        out_type=[
            jax.ShapeDtypeStruct((NC, N, F), jnp.float32),
            jax.ShapeDtypeStruct((N, F), jnp.float32),
        ],
        mesh=mesh,
        compiler_params=pltpu.CompilerParams(use_tc_tiling_on_sc=False),
        scratch_types=[
            pltpu.VMEM((CE, F, F), jnp.float32),
            pltpu.VMEM((CE, F), jnp.float32),
            pltpu.VMEM((CE, F), jnp.float32),
            pltpu.VMEM((CE, F), jnp.float32),
            pltpu.VMEM((CE,), jnp.int32),
            pltpu.VMEM((CE,), jnp.int32),
            pltpu.VMEM((TAIL_E,), jnp.int32),
            pltpu.VMEM_SHARED((N, F), jnp.float32),
        ],
    )


def kernel(feat, loop_weight, W, m_bias, h_bias, edge_index):
    src = edge_index[0]
    dst = edge_index[1]
    mb = m_bias.reshape(E, F)
    hb = h_bias.reshape(N, F)
    sc = _make_sc_call()
    p, loop_out = sc(feat, loop_weight, W, mb, hb, src, dst)
    return p[0] + p[1] + loop_out


# speedup vs baseline: 1.0432x; 1.0432x over previous
import jax
import jax.numpy as jnp
from jax import lax
from jax.experimental import pallas as pl
from jax.experimental.pallas import tpu as pltpu
from jax.experimental.pallas import tpu_sc as plsc

N = 10000
E = 160000
F = 16

NC = 2
NS = 16
NW = NC * NS

EPT = E // NW
CE = 128
NFULL = EPT // CE
TAIL_E = EPT - NFULL * CE

CN = 128
NJ_FULL = N // CN
TAIL_N = N - NJ_FULL * CN

ROWS_A = 624
ROWS_LAST = N - (NS - 1) * ROWS_A

_GDN = lax.GatherDimensionNumbers(
    offset_dims=(), collapsed_slice_dims=(0,), start_index_map=(0,))


def _bcast_lane(x_row, i):
    idx = jnp.full((F, 1), i, dtype=jnp.int32)
    return lax.gather(x_row, idx, dimension_numbers=_GDN, slice_sizes=(1,),
                      mode=lax.GatherScatterMode.PROMISE_IN_BOUNDS)


def _compute_chunk(cnt, x_ref, w_ref, b_ref, m_ref):
    def body(e):
        x_row = x_ref[e, :]
        a0 = b_ref[e, :]
        a1 = _bcast_lane(x_row, 1) * w_ref[e, 1, :]
        a2 = _bcast_lane(x_row, 2) * w_ref[e, 2, :]
        a3 = _bcast_lane(x_row, 3) * w_ref[e, 3, :]
        a0 = a0 + _bcast_lane(x_row, 0) * w_ref[e, 0, :]
        for i in range(4, F, 4):
            a0 = a0 + _bcast_lane(x_row, i) * w_ref[e, i, :]
            a1 = a1 + _bcast_lane(x_row, i + 1) * w_ref[e, i + 1, :]
            a2 = a2 + _bcast_lane(x_row, i + 2) * w_ref[e, i + 2, :]
            a3 = a3 + _bcast_lane(x_row, i + 3) * w_ref[e, i + 3, :]
        m_ref[e, :] = (a0 + a1) + (a2 + a3)
    plsc.parallel_loop(0, cnt, 1, unroll=2)(body)


def _gnn_body(feat, lw, w_hbm, mb, hb, src_h, dst_h, p_out, loop_out,
              w_buf, x_buf, b_buf, m_buf, sidx, didx, didx_t, shared):
    cid = lax.axis_index("c")
    sid = lax.axis_index("s")
    wid = cid * NS + sid

    row0 = pl.multiple_of(sid * ROWS_A, 8)

    def zbody(i, _):
        m_buf[i, :] = jnp.zeros((F,), jnp.float32)
        return 0
    lax.fori_loop(0, CE, zbody, 0)
    for k in range(4):
        pltpu.sync_copy(m_buf.at[pl.ds(0, CE)],
                        shared.at[pl.ds(pl.multiple_of(row0 + k * CE, 8), CE)])

    @pl.when(sid < NS - 1)
    def _zero_tail_a():
        pltpu.sync_copy(m_buf.at[pl.ds(0, ROWS_A - 4 * CE)],
                        shared.at[pl.ds(pl.multiple_of(row0 + 4 * CE, 8),
                                        ROWS_A - 4 * CE)])

    @pl.when(sid == NS - 1)
    def _zero_tail_b():
        pltpu.sync_copy(m_buf.at[pl.ds(0, CE)],
                        shared.at[pl.ds(pl.multiple_of(row0 + 4 * CE, 8), CE)])

    plsc.subcore_barrier()

    def loop_chunk(noff, cnt):
        pltpu.sync_copy(lw.at[pl.ds(noff, cnt)], w_buf.at[pl.ds(0, cnt)])
        pltpu.sync_copy(feat.at[pl.ds(noff, cnt)], x_buf.at[pl.ds(0, cnt)])
        pltpu.sync_copy(hb.at[pl.ds(noff, cnt)], b_buf.at[pl.ds(0, cnt)])
        _compute_chunk(cnt, x_buf, w_buf, b_buf, m_buf)
        pltpu.sync_copy(m_buf.at[pl.ds(0, cnt)], loop_out.at[pl.ds(noff, cnt)])

    @pl.when(cid == 0)
    def _loop_term():
        for k in range(5):
            j = sid + NS * k

            @pl.when(j < NJ_FULL)
            def _one():
                loop_chunk(pl.multiple_of(j * CN, 8), CN)

        @pl.when(sid == NS - 1)
        def _node_tail():
            loop_chunk(NJ_FULL * CN, TAIL_N)

    ebase = wid * EPT

    def edge_chunk(off, cnt, didx_ref):
        off = pl.multiple_of(off, 8)
        pltpu.sync_copy(src_h.at[pl.ds(off, cnt)], sidx.at[pl.ds(0, cnt)])
        pltpu.sync_copy(dst_h.at[pl.ds(off, cnt)], didx_ref)
        pltpu.sync_copy(w_hbm.at[pl.ds(off, cnt)], w_buf.at[pl.ds(0, cnt)])
        pltpu.sync_copy(mb.at[pl.ds(off, cnt)], b_buf.at[pl.ds(0, cnt)])
        pltpu.sync_copy(feat.at[sidx.at[pl.ds(0, cnt)]],
                        x_buf.at[pl.ds(0, cnt)])
        _compute_chunk(cnt, x_buf, w_buf, b_buf, m_buf)
        pltpu.sync_copy(m_buf.at[pl.ds(0, cnt)], shared.at[didx_ref],
                        add=True)

    def echunk_body(c, _):
        edge_chunk(ebase + c * CE, CE, didx)
        return 0
    lax.fori_loop(0, NFULL, echunk_body, 0)
    edge_chunk(ebase + NFULL * CE, TAIL_E, didx_t)

    plsc.subcore_barrier()

    @pl.when(sid < NS - 1)
    def _pub_a():
        pltpu.sync_copy(shared.at[pl.ds(row0, ROWS_A)],
                        p_out.at[cid, pl.ds(row0, ROWS_A)])

    @pl.when(sid == NS - 1)
    def _pub_b():
        pltpu.sync_copy(shared.at[pl.ds(row0, ROWS_LAST)],
                        p_out.at[cid, pl.ds(row0, ROWS_LAST)])


def _make_sc_call():
    mesh = plsc.VectorSubcoreMesh(core_axis_name="c", subcore_axis_name="s")
    return pl.kernel(
        _gnn_body,
        out_type=[
            jax.ShapeDtypeStruct((NC, N, F), jnp.float32),
            jax.ShapeDtypeStruct((N, F), jnp.float32),
        ],
        mesh=mesh,
        compiler_params=pltpu.CompilerParams(use_tc_tiling_on_sc=False),
        scratch_types=[
            pltpu.VMEM((CE, F, F), jnp.float32),
            pltpu.VMEM((CE, F), jnp.float32),
            pltpu.VMEM((CE, F), jnp.float32),
            pltpu.VMEM((CE, F), jnp.float32),
            pltpu.VMEM((CE,), jnp.int32),
            pltpu.VMEM((CE,), jnp.int32),
            pltpu.VMEM((TAIL_E,), jnp.int32),
            pltpu.VMEM_SHARED((N, F), jnp.float32),
        ],
    )


def kernel(feat, loop_weight, W, m_bias, h_bias, edge_index):
    src = edge_index[0]
    dst = edge_index[1]
    mb = m_bias.reshape(E, F)
    hb = h_bias.reshape(N, F)
    sc = _make_sc_call()
    p, loop_out = sc(feat, loop_weight, W, mb, hb, src, dst)
    return p[0] + p[1] + loop_out


# speedup vs baseline: 1.1113x; 1.0653x over previous
import jax
import jax.numpy as jnp
from jax import lax
from jax.experimental import pallas as pl
from jax.experimental.pallas import tpu as pltpu
from jax.experimental.pallas import tpu_sc as plsc

N = 10000
E = 160000
F = 16

NC = 2
NS = 16
NW = NC * NS

EPT = E // NW
CE = 128
NFULL = EPT // CE
TAIL_E = EPT - NFULL * CE

CN = 128
NJ_FULL = N // CN
TAIL_N = N - NJ_FULL * CN

ROWS_A = 624
ROWS_LAST = N - (NS - 1) * ROWS_A

_GDN = lax.GatherDimensionNumbers(
    offset_dims=(), collapsed_slice_dims=(0,), start_index_map=(0,))


def _bcast_lane(x_row, i):
    idx = jnp.full((F, 1), i, dtype=jnp.int32)
    return lax.gather(x_row, idx, dimension_numbers=_GDN, slice_sizes=(1,),
                      mode=lax.GatherScatterMode.PROMISE_IN_BOUNDS)


def _compute_chunk(cnt, x_ref, w_ref, b_ref, m_ref):
    def body(e):
        x_row = x_ref[e, :]
        a0 = b_ref[e, :]
        a1 = _bcast_lane(x_row, 1) * w_ref[e, 1, :]
        a2 = _bcast_lane(x_row, 2) * w_ref[e, 2, :]
        a3 = _bcast_lane(x_row, 3) * w_ref[e, 3, :]
        a0 = a0 + _bcast_lane(x_row, 0) * w_ref[e, 0, :]
        for i in range(4, F, 4):
            a0 = a0 + _bcast_lane(x_row, i) * w_ref[e, i, :]
            a1 = a1 + _bcast_lane(x_row, i + 1) * w_ref[e, i + 1, :]
            a2 = a2 + _bcast_lane(x_row, i + 2) * w_ref[e, i + 2, :]
            a3 = a3 + _bcast_lane(x_row, i + 3) * w_ref[e, i + 3, :]
        m_ref[e, :] = (a0 + a1) + (a2 + a3)
    plsc.parallel_loop(0, cnt, 1, unroll=2)(body)


def _gnn_body(feat, lw, w_hbm, mb, hb, src_h, dst_h, p_out, loop_out,
              w_buf, x_buf, b_buf, m_buf, sidx, didx, didx_t, shared,
              sem_i, sem_a, sem_g):
    cid = lax.axis_index("c")
    sid = lax.axis_index("s")
    wid = cid * NS + sid

    row0 = pl.multiple_of(sid * ROWS_A, 8)

    def zbody(i, _):
        m_buf[i, :] = jnp.zeros((F,), jnp.float32)
        return 0
    lax.fori_loop(0, CE, zbody, 0)
    for k in range(4):
        pltpu.sync_copy(m_buf.at[pl.ds(0, CE)],
                        shared.at[pl.ds(pl.multiple_of(row0 + k * CE, 8), CE)])

    @pl.when(sid < NS - 1)
    def _zero_tail_a():
        pltpu.sync_copy(m_buf.at[pl.ds(0, ROWS_A - 4 * CE)],
                        shared.at[pl.ds(pl.multiple_of(row0 + 4 * CE, 8),
                                        ROWS_A - 4 * CE)])

    @pl.when(sid == NS - 1)
    def _zero_tail_b():
        pltpu.sync_copy(m_buf.at[pl.ds(0, CE)],
                        shared.at[pl.ds(pl.multiple_of(row0 + 4 * CE, 8), CE)])

    plsc.subcore_barrier()

    def loop_chunk(noff, cnt):
        d1 = pltpu.async_copy(lw.at[pl.ds(noff, cnt)],
                              w_buf.at[pl.ds(0, cnt)], sem_a)
        d2 = pltpu.async_copy(feat.at[pl.ds(noff, cnt)],
                              x_buf.at[pl.ds(0, cnt)], sem_a)
        d3 = pltpu.async_copy(hb.at[pl.ds(noff, cnt)],
                              b_buf.at[pl.ds(0, cnt)], sem_a)
        d1.wait(); d2.wait(); d3.wait()
        _compute_chunk(cnt, x_buf, w_buf, b_buf, m_buf)
        pltpu.sync_copy(m_buf.at[pl.ds(0, cnt)], loop_out.at[pl.ds(noff, cnt)])

    @pl.when(cid == 0)
    def _loop_term():
        for k in range(5):
            j = sid + NS * k

            @pl.when(j < NJ_FULL)
            def _one():
                loop_chunk(pl.multiple_of(j * CN, 8), CN)

        @pl.when(sid == NS - 1)
        def _node_tail():
            loop_chunk(NJ_FULL * CN, TAIL_N)

    ebase = wid * EPT

    def edge_chunk(off, cnt, didx_ref):
        off = pl.multiple_of(off, 8)
        di = pltpu.async_copy(src_h.at[pl.ds(off, cnt)],
                              sidx.at[pl.ds(0, cnt)], sem_i)
        d2 = pltpu.async_copy(dst_h.at[pl.ds(off, cnt)], didx_ref, sem_a)
        d3 = pltpu.async_copy(w_hbm.at[pl.ds(off, cnt)],
                              w_buf.at[pl.ds(0, cnt)], sem_a)
        d4 = pltpu.async_copy(mb.at[pl.ds(off, cnt)],
                              b_buf.at[pl.ds(0, cnt)], sem_a)
        di.wait()
        dg = pltpu.async_copy(feat.at[sidx.at[pl.ds(0, cnt)]],
                              x_buf.at[pl.ds(0, cnt)], sem_g)
        d2.wait(); d3.wait(); d4.wait(); dg.wait()
        _compute_chunk(cnt, x_buf, w_buf, b_buf, m_buf)
        pltpu.sync_copy(m_buf.at[pl.ds(0, cnt)], shared.at[didx_ref],
                        add=True)

    def echunk_body(c, _):
        edge_chunk(ebase + c * CE, CE, didx)
        return 0
    lax.fori_loop(0, NFULL, echunk_body, 0)
    edge_chunk(ebase + NFULL * CE, TAIL_E, didx_t)

    plsc.subcore_barrier()

    @pl.when(sid < NS - 1)
    def _pub_a():
        pltpu.sync_copy(shared.at[pl.ds(row0, ROWS_A)],
                        p_out.at[cid, pl.ds(row0, ROWS_A)])

    @pl.when(sid == NS - 1)
    def _pub_b():
        pltpu.sync_copy(shared.at[pl.ds(row0, ROWS_LAST)],
                        p_out.at[cid, pl.ds(row0, ROWS_LAST)])


def _make_sc_call():
    mesh = plsc.VectorSubcoreMesh(core_axis_name="c", subcore_axis_name="s")
    return pl.kernel(
        _gnn_body,
        out_type=[
            jax.ShapeDtypeStruct((NC, N, F), jnp.float32),
            jax.ShapeDtypeStruct((N, F), jnp.float32),
        ],
        mesh=mesh,
        compiler_params=pltpu.CompilerParams(use_tc_tiling_on_sc=False),
        scratch_types=[
            pltpu.VMEM((CE, F, F), jnp.float32),
            pltpu.VMEM((CE, F), jnp.float32),
            pltpu.VMEM((CE, F), jnp.float32),
            pltpu.VMEM((CE, F), jnp.float32),
            pltpu.VMEM((CE,), jnp.int32),
            pltpu.VMEM((CE,), jnp.int32),
            pltpu.VMEM((TAIL_E,), jnp.int32),
            pltpu.VMEM_SHARED((N, F), jnp.float32),
            pltpu.SemaphoreType.DMA,
            pltpu.SemaphoreType.DMA,
            pltpu.SemaphoreType.DMA,
        ],
    )


def kernel(feat, loop_weight, W, m_bias, h_bias, edge_index):
    src = edge_index[0]
    dst = edge_index[1]
    mb = m_bias.reshape(E, F)
    hb = h_bias.reshape(N, F)
    sc = _make_sc_call()
    p, loop_out = sc(feat, loop_weight, W, mb, hb, src, dst)
    return p[0] + p[1] + loop_out


# speedup vs baseline: 1.1680x; 1.0511x over previous
import jax
import jax.numpy as jnp
from jax import lax
from jax.experimental import pallas as pl
from jax.experimental.pallas import tpu as pltpu
from jax.experimental.pallas import tpu_sc as plsc

N = 10000
E = 160000
F = 16

NC = 2
NS = 16
NW = NC * NS

EPT = E // NW
CE = 128
NFULL = EPT // CE
TAIL_E = EPT - NFULL * CE

CN = 128
NJ_FULL = N // CN
TAIL_N = N - NJ_FULL * CN

ROWS_A = 624
ROWS_LAST = N - (NS - 1) * ROWS_A

_GDN = lax.GatherDimensionNumbers(
    offset_dims=(), collapsed_slice_dims=(0,), start_index_map=(0,))


def _bcast_lane(x_row, i):
    idx = jnp.full((F, 1), i, dtype=jnp.int32)
    return lax.gather(x_row, idx, dimension_numbers=_GDN, slice_sizes=(1,),
                      mode=lax.GatherScatterMode.PROMISE_IN_BOUNDS)


def _compute_chunk(cnt, x_ref, w_ref, b_ref, m_ref):
    def body(e):
        x_row = x_ref[e, :]
        a0 = b_ref[e, :]
        a1 = _bcast_lane(x_row, 1) * w_ref[e, 1, :]
        a2 = _bcast_lane(x_row, 2) * w_ref[e, 2, :]
        a3 = _bcast_lane(x_row, 3) * w_ref[e, 3, :]
        a0 = a0 + _bcast_lane(x_row, 0) * w_ref[e, 0, :]
        for i in range(4, F, 4):
            a0 = a0 + _bcast_lane(x_row, i) * w_ref[e, i, :]
            a1 = a1 + _bcast_lane(x_row, i + 1) * w_ref[e, i + 1, :]
            a2 = a2 + _bcast_lane(x_row, i + 2) * w_ref[e, i + 2, :]
            a3 = a3 + _bcast_lane(x_row, i + 3) * w_ref[e, i + 3, :]
        m_ref[e, :] = (a0 + a1) + (a2 + a3)
    plsc.parallel_loop(0, cnt, 1, unroll=2)(body)


def _gnn_body(feat, lw, w_hbm, mb, hb, src_h, dst_h, p_out, loop_out,
              w_buf, x_buf, b_buf, m_buf, sidx, didx, didx_s, didx_t, shared,
              si0, si1, sa0, sa1, sg0, sg1, ss0, ss1):
    cid = lax.axis_index("c")
    sid = lax.axis_index("s")
    wid = cid * NS + sid
    si = (si0, si1)
    sa = (sa0, sa1)
    sg = (sg0, sg1)
    ss = (ss0, ss1)

    row0 = pl.multiple_of(sid * ROWS_A, 8)

    def zbody(i, _):
        m_buf[0, i, :] = jnp.zeros((F,), jnp.float32)
        return 0
    lax.fori_loop(0, CE, zbody, 0)
    for k in range(4):
        pltpu.sync_copy(m_buf.at[0],
                        shared.at[pl.ds(pl.multiple_of(row0 + k * CE, 8), CE)])

    @pl.when(sid < NS - 1)
    def _zero_tail_a():
        pltpu.sync_copy(m_buf.at[0, pl.ds(0, ROWS_A - 4 * CE)],
                        shared.at[pl.ds(pl.multiple_of(row0 + 4 * CE, 8),
                                        ROWS_A - 4 * CE)])

    @pl.when(sid == NS - 1)
    def _zero_tail_b():
        pltpu.sync_copy(m_buf.at[0],
                        shared.at[pl.ds(pl.multiple_of(row0 + 4 * CE, 8), CE)])

    plsc.subcore_barrier()

    def loop_chunk(noff, cnt):
        d1 = pltpu.async_copy(lw.at[pl.ds(noff, cnt)],
                              w_buf.at[0, pl.ds(0, cnt)], sa0)
        d2 = pltpu.async_copy(feat.at[pl.ds(noff, cnt)],
                              x_buf.at[0, pl.ds(0, cnt)], sa0)
        d3 = pltpu.async_copy(hb.at[pl.ds(noff, cnt)],
                              b_buf.at[0, pl.ds(0, cnt)], sa0)
        d1.wait(); d2.wait(); d3.wait()
        _compute_chunk(cnt, x_buf.at[0], w_buf.at[0], b_buf.at[0],
                       m_buf.at[0])
        pltpu.sync_copy(m_buf.at[0, pl.ds(0, cnt)],
                        loop_out.at[pl.ds(noff, cnt)])

    @pl.when(cid == 0)
    def _loop_term():
        for k in range(5):
            j = sid + NS * k

            @pl.when(j < NJ_FULL)
            def _one():
                loop_chunk(pl.multiple_of(j * CN, 8), CN)

        @pl.when(sid == NS - 1)
        def _node_tail():
            loop_chunk(NJ_FULL * CN, TAIL_N)

    ebase = wid * EPT

    def eoff(c):
        return pl.multiple_of(ebase + c * CE, 8)

    def issue_idx(c, p):
        pltpu.async_copy(src_h.at[pl.ds(eoff(c), CE)], sidx.at[p], si[p])
        pltpu.async_copy(dst_h.at[pl.ds(eoff(c), CE)], didx.at[p], si[p])

    def wait_idx(p):
        pltpu.make_async_copy(src_h.at[pl.ds(0, CE)], sidx.at[p],
                              si[p]).wait()
        pltpu.make_async_copy(dst_h.at[pl.ds(0, CE)], didx.at[p],
                              si[p]).wait()

    def issue_gather(p):
        pltpu.async_copy(feat.at[sidx.at[p]], x_buf.at[p], sg[p])

    def wait_gather(p):
        pltpu.make_async_copy(mb.at[pl.ds(0, CE)], x_buf.at[p],
                              sg[p]).wait()

    def issue_bulk(c, p):
        pltpu.async_copy(w_hbm.at[pl.ds(eoff(c), CE)], w_buf.at[p], sa[p])
        pltpu.async_copy(mb.at[pl.ds(eoff(c), CE)], b_buf.at[p], sa[p])

    def wait_bulk(p):
        pltpu.make_async_copy(w_hbm.at[pl.ds(0, CE)], w_buf.at[p],
                              sa[p]).wait()
        pltpu.make_async_copy(mb.at[pl.ds(0, CE)], b_buf.at[p],
                              sa[p]).wait()

    def issue_scatter(p):
        pltpu.async_copy(m_buf.at[p], shared.at[didx_s.at[p]], ss[p],
                         add=True)

    def wait_scatter(p):
        pltpu.make_async_copy(mb.at[pl.ds(0, CE)], m_buf.at[p],
                              ss[p]).wait()

    def save_didx(p):
        for k in range(CE // F):
            didx_s[p, pl.ds(k * F, F)] = didx[p, pl.ds(k * F, F)]

    def stage(c, p, first_pair=None, has_next=True, wait_prev_scatter=True,
              prefetch2=None):
        q = 1 - p
        if has_next:
            wait_idx(q)
            issue_gather(q)
            issue_bulk_c1 = c + 1
            issue_bulk(issue_bulk_c1, q)
        if wait_prev_scatter:
            wait_scatter(p)
        wait_gather(p)
        wait_bulk(p)
        save_didx(p)
        if prefetch2 is not None:
            issue_idx(prefetch2, p)
        _compute_chunk(CE, x_buf.at[p], w_buf.at[p], b_buf.at[p],
                       m_buf.at[p])
        issue_scatter(p)

    issue_idx(0, 0)
    issue_idx(1, 1)
    wait_idx(0)
    issue_gather(0)
    issue_bulk(0, 0)

    stage(0, 0, wait_prev_scatter=False, prefetch2=2)
    stage(1, 1, wait_prev_scatter=False, prefetch2=3)

    def pair_body(t, _):
        ce = 2 + 2 * t
        stage(ce, 0, prefetch2=ce + 2)
        stage(ce + 1, 1, prefetch2=ce + 3)
        return 0
    lax.fori_loop(0, (NFULL - 5) // 2, pair_body, 0)

    stage(NFULL - 3, 0, prefetch2=NFULL - 1)
    stage(NFULL - 2, 1, prefetch2=None)
    stage(NFULL - 1, 0, has_next=False, prefetch2=None)

    wait_scatter(1)
    wait_scatter(0)

    toff = pl.multiple_of(ebase + NFULL * CE, 8)
    pltpu.sync_copy(src_h.at[pl.ds(toff, TAIL_E)], sidx.at[0, pl.ds(0, TAIL_E)])
    pltpu.sync_copy(dst_h.at[pl.ds(toff, TAIL_E)], didx_t)
    pltpu.sync_copy(w_hbm.at[pl.ds(toff, TAIL_E)], w_buf.at[0, pl.ds(0, TAIL_E)])
    pltpu.sync_copy(mb.at[pl.ds(toff, TAIL_E)], b_buf.at[0, pl.ds(0, TAIL_E)])
    pltpu.sync_copy(feat.at[sidx.at[0, pl.ds(0, TAIL_E)]],
                    x_buf.at[0, pl.ds(0, TAIL_E)])
    _compute_chunk(TAIL_E, x_buf.at[0], w_buf.at[0], b_buf.at[0], m_buf.at[0])
    pltpu.sync_copy(m_buf.at[0, pl.ds(0, TAIL_E)], shared.at[didx_t], add=True)

    plsc.subcore_barrier()

    @pl.when(sid < NS - 1)
    def _pub_a():
        pltpu.sync_copy(shared.at[pl.ds(row0, ROWS_A)],
                        p_out.at[cid, pl.ds(row0, ROWS_A)])

    @pl.when(sid == NS - 1)
    def _pub_b():
        pltpu.sync_copy(shared.at[pl.ds(row0, ROWS_LAST)],
                        p_out.at[cid, pl.ds(row0, ROWS_LAST)])


def _make_sc_call():
    mesh = plsc.VectorSubcoreMesh(core_axis_name="c", subcore_axis_name="s")
    return pl.kernel(
        _gnn_body,
        out_type=[
            jax.ShapeDtypeStruct((NC, N, F), jnp.float32),
            jax.ShapeDtypeStruct((N, F), jnp.float32),
        ],
        mesh=mesh,
        compiler_params=pltpu.CompilerParams(use_tc_tiling_on_sc=False),
        scratch_types=[
            pltpu.VMEM((2, CE, F, F), jnp.float32),
            pltpu.VMEM((2, CE, F), jnp.float32),
            pltpu.VMEM((2, CE, F), jnp.float32),
            pltpu.VMEM((2, CE, F), jnp.float32),
            pltpu.VMEM((2, CE), jnp.int32),
            pltpu.VMEM((2, CE), jnp.int32),
            pltpu.VMEM((2, CE), jnp.int32),
            pltpu.VMEM((TAIL_E,), jnp.int32),
            pltpu.VMEM_SHARED((N, F), jnp.float32),
            pltpu.SemaphoreType.DMA,
            pltpu.SemaphoreType.DMA,
            pltpu.SemaphoreType.DMA,
            pltpu.SemaphoreType.DMA,
            pltpu.SemaphoreType.DMA,
            pltpu.SemaphoreType.DMA,
            pltpu.SemaphoreType.DMA,
            pltpu.SemaphoreType.DMA,
        ],
    )


def kernel(feat, loop_weight, W, m_bias, h_bias, edge_index):
    src = edge_index[0]
    dst = edge_index[1]
    mb = m_bias.reshape(E, F)
    hb = h_bias.reshape(N, F)
    sc = _make_sc_call()
    p, loop_out = sc(feat, loop_weight, W, mb, hb, src, dst)
    return p[0] + p[1] + loop_out


# speedup vs baseline: 6.9325x; 5.9352x over previous
import jax
import jax.numpy as jnp
from jax import lax
from jax.experimental import pallas as pl
from jax.experimental.pallas import tpu as pltpu
from jax.experimental.pallas import tpu_sc as plsc

N = 10000
E = 160000
F = 16

NC = 2
NS = 16
NW = NC * NS

CE = 128
NCH_E = E // CE
CPT = NCH_E // NW
NPAD = 10240
NCH_N = NPAD // CE

EPT = E // NW
NFULL = EPT // CE
TAIL_E = EPT - NFULL * CE

ROWS_A = 624
ROWS_LAST = N - (NS - 1) * ROWS_A

_GDN = lax.GatherDimensionNumbers(
    offset_dims=(), collapsed_slice_dims=(0,), start_index_map=(0,))


def _gather16(x, idx):
    return lax.gather(x, idx.reshape(F, 1), dimension_numbers=_GDN,
                      slice_sizes=(1,),
                      mode=lax.GatherScatterMode.PROMISE_IN_BOUNDS)


def _iota16():
    return lax.iota(jnp.int32, F)


def _transpose16(v):
    iota = _iota16()
    for s in range(4):
        d = 1 << s
        idx = iota ^ d
        mask = (iota & d) == 0
        nv = list(v)
        for j in range(F):
            if j & d == 0:
                a, b = v[j], v[j | d]
                nv[j] = jnp.where(mask, a, _gather16(b, idx))
                nv[j | d] = jnp.where(mask, _gather16(a, idx), b)
        v = nv
    return v


def _compute_chunk_t(x_ref, w_ref, b_ref, m_ref):
    iota16 = _iota16() * F

    def blk(b_i):
        xv = [x_ref[b_i * F + j, pl.ds(0, F)] for j in range(F)]
        xt = _transpose16(xv)
        col = pl.ds(pl.multiple_of(b_i * F, 16), F)
        base = b_i * (F * F)
        for o in range(F):
            a0 = b_ref[o, col]
            a1 = xt[1] * w_ref[1, o, col]
            a2 = xt[2] * w_ref[2, o, col]
            a3 = xt[3] * w_ref[3, o, col]
            a0 = a0 + xt[0] * w_ref[0, o, col]
            for i in range(4, F, 4):
                a0 = a0 + xt[i] * w_ref[i, o, col]
                a1 = a1 + xt[i + 1] * w_ref[i + 1, o, col]
                a2 = a2 + xt[i + 2] * w_ref[i + 2, o, col]
                a3 = a3 + xt[i + 3] * w_ref[i + 3, o, col]
            acc = (a0 + a1) + (a2 + a3)
            idxv = iota16 + (base + o)
            plsc.store_scatter(m_ref, [idxv], acc)
    plsc.parallel_loop(0, CE // F, 1)(blk)


def _msg_body(wt, mbt, lwt, hbt, featp, src_h, msg_f, loop_f,
              w_buf, x_buf, b_buf, m0, m1, sidx0, sidx1,
              si0, si1, sa0, sa1, sg0, sg1, so0, so1):
    cid = lax.axis_index("c")
    sid = lax.axis_index("s")
    wid = cid * NS + sid
    si = (si0, si1)
    sa = (sa0, sa1)
    sg = (sg0, sg1)
    so = (so0, so1)
    m1d = (m0, m1)
    sidx = (sidx0, sidx1)

    cb = wid * CPT

    def c128(ch):
        return pl.multiple_of(ch * CE, 128)

    def issue_idx(ch, p):
        pltpu.async_copy(src_h.at[pl.ds(c128(ch), CE)], sidx[p], si[p])

    def wait_idx(p):
        pltpu.make_async_copy(src_h.at[pl.ds(0, CE)], sidx[p], si[p]).wait()

    def issue_gather(p):
        pltpu.async_copy(featp.at[sidx[p]], x_buf.at[p], sg[p])

    def wait_gather(p):
        pltpu.make_async_copy(featp.at[pl.ds(0, CE)], x_buf.at[p],
                              sg[p]).wait()

    def issue_bulk(ch, p, w_src, b_src):
        pltpu.async_copy(w_src.at[:, :, pl.ds(c128(ch), CE)],
                         w_buf.at[p], sa[p])
        pltpu.async_copy(b_src.at[:, pl.ds(c128(ch), CE)],
                         b_buf.at[p], sa[p])

    def wait_bulk(p):
        pltpu.make_async_copy(wt.at[:, :, pl.ds(0, CE)], w_buf.at[p],
                              sa[p]).wait()
        pltpu.make_async_copy(mbt.at[:, pl.ds(0, CE)], b_buf.at[p],
                              sa[p]).wait()

    def issue_out(ch, p):
        pltpu.async_copy(
            m1d[p],
            msg_f.at[pl.ds(pl.multiple_of(ch * (CE * F), 8), CE * F)],
            so[p])

    def wait_out(p):
        pltpu.make_async_copy(msg_f.at[pl.ds(0, CE * F)], m1d[p],
                              so[p]).wait()

    def stage_g(c, p):
        q = 1 - p
        wait_idx(q)
        issue_gather(q)
        issue_bulk(cb + c + 1, q, wt, mbt)

        @pl.when(c >= 2)
        def _drain_out():
            wait_out(p)
        wait_gather(p)
        wait_bulk(p)

        @pl.when(c + 2 <= CPT - 1)
        def _prefetch():
            issue_idx(cb + c + 2, p)
        _compute_chunk_t(x_buf.at[p], w_buf.at[p], b_buf.at[p], m1d[p])
        issue_out(cb + c, p)

    issue_idx(cb + 0, 0)
    issue_idx(cb + 1, 1)
    wait_idx(0)
    issue_gather(0)
    issue_bulk(cb, 0, wt, mbt)

    def pair_body(t, _):
        ce = 2 * t
        stage_g(ce, 0)
        stage_g(ce + 1, 1)
        return 0
    lax.fori_loop(0, (CPT - 1) // 2, pair_body, 0)

    wait_out(0)
    wait_gather(0)
    wait_bulk(0)
    _compute_chunk_t(x_buf.at[0], w_buf.at[0], b_buf.at[0], m1d[0])
    issue_out(cb + CPT - 1, 0)
    wait_out(1)
    wait_out(0)

    def sync_edge_chunk(ch):
        pltpu.sync_copy(src_h.at[pl.ds(c128(ch), CE)], sidx[0])
        d1 = pltpu.async_copy(featp.at[sidx[0]], x_buf.at[0], sg[0])
        d2 = pltpu.async_copy(wt.at[:, :, pl.ds(c128(ch), CE)],
                              w_buf.at[0], sa[0])
        d3 = pltpu.async_copy(mbt.at[:, pl.ds(c128(ch), CE)],
                              b_buf.at[0], sa[0])
        d1.wait(); d2.wait(); d3.wait()
        _compute_chunk_t(x_buf.at[0], w_buf.at[0], b_buf.at[0], m1d[0])
        pltpu.sync_copy(
            m1d[0],
            msg_f.at[pl.ds(pl.multiple_of(ch * (CE * F), 8), CE * F)])

    @pl.when(wid < NCH_E - NW * CPT)
    def _leftover_edges():
        sync_edge_chunk(NW * CPT + wid)

    def sync_node_chunk(ch):
        d0 = pltpu.async_copy(featp.at[pl.ds(c128(ch), CE)],
                              x_buf.at[0], sg[0])
        d2 = pltpu.async_copy(lwt.at[:, :, pl.ds(c128(ch), CE)],
                              w_buf.at[0], sa[0])
        d3 = pltpu.async_copy(hbt.at[:, pl.ds(c128(ch), CE)],
                              b_buf.at[0], sa[0])
        d0.wait(); d2.wait(); d3.wait()
        _compute_chunk_t(x_buf.at[0], w_buf.at[0], b_buf.at[0], m1d[0])
        pltpu.sync_copy(
            m1d[0],
            loop_f.at[pl.ds(pl.multiple_of(ch * (CE * F), 8), CE * F)])

    node_cnt = jnp.where(wid < NCH_N - 2 * NW, 3, 2)

    def node_body(k, _):
        ch = jnp.where(k < 2, 2 * wid + k, 2 * NW + wid)
        sync_node_chunk(ch)
        return 0
    lax.fori_loop(0, node_cnt, node_body, 0)


def _scatter_body(msg2, dst_h, p_out,
                  m_buf, m_s, didx, didx_s, didx_t, shared,
                  si0, si1, ss0, ss1):
    cid = lax.axis_index("c")
    sid = lax.axis_index("s")
    wid = cid * NS + sid
    si = (si0, si1)
    ss = (ss0, ss1)

    row0 = pl.multiple_of(sid * ROWS_A, 8)

    def zbody(i, _):
        m_buf[0, i, :] = jnp.zeros((F,), jnp.float32)
        return 0
    lax.fori_loop(0, CE, zbody, 0)
    for k in range(4):
        pltpu.sync_copy(m_buf.at[0],
                        shared.at[pl.ds(pl.multiple_of(row0 + k * CE, 8), CE)])

    @pl.when(sid < NS - 1)
    def _zero_tail_a():
        pltpu.sync_copy(m_buf.at[0, pl.ds(0, ROWS_A - 4 * CE)],
                        shared.at[pl.ds(pl.multiple_of(row0 + 4 * CE, 8),
                                        ROWS_A - 4 * CE)])

    @pl.when(sid == NS - 1)
    def _zero_tail_b():
        pltpu.sync_copy(m_buf.at[0],
                        shared.at[pl.ds(pl.multiple_of(row0 + 4 * CE, 8), CE)])

    plsc.subcore_barrier()

    ebase = wid * EPT

    def eoff(c):
        return pl.multiple_of(ebase + c * CE, 8)

    def issue_in(c, p):
        pltpu.async_copy(dst_h.at[pl.ds(eoff(c), CE)], didx.at[p], si[p])
        pltpu.async_copy(msg2.at[pl.ds(eoff(c), CE)], m_buf.at[p], si[p])

    def wait_in(p):
        pltpu.make_async_copy(dst_h.at[pl.ds(0, CE)], didx.at[p],
                              si[p]).wait()
        pltpu.make_async_copy(msg2.at[pl.ds(0, CE)], m_buf.at[p],
                              si[p]).wait()

    def issue_scatter(p):
        pltpu.async_copy(m_s.at[p], shared.at[didx_s.at[p]], ss[p],
                         add=True)

    def wait_scatter(p):
        pltpu.make_async_copy(msg2.at[pl.ds(0, CE)], m_s.at[p],
                              ss[p]).wait()

    def save(p):
        for k in range(CE // F):
            didx_s[p, pl.ds(k * F, F)] = didx[p, pl.ds(k * F, F)]

        def cp(i, _):
            m_s[p, i, :] = m_buf[p, i, :]
            return 0
        lax.fori_loop(0, CE, cp, 0)

    def stage(c, p, wait_prev_scatter=True, prefetch2=None):
        if wait_prev_scatter:
            wait_scatter(p)
        wait_in(p)
        save(p)
        if prefetch2 is not None:
            issue_in(prefetch2, p)
        issue_scatter(p)

    issue_in(0, 0)
    issue_in(1, 1)
    stage(0, 0, wait_prev_scatter=False, prefetch2=2)
    stage(1, 1, wait_prev_scatter=False, prefetch2=3)

    def pair_body(t, _):
        ce = 2 + 2 * t
        stage(ce, 0, prefetch2=ce + 2)
        stage(ce + 1, 1, prefetch2=ce + 3)
        return 0
    lax.fori_loop(0, (NFULL - 5) // 2, pair_body, 0)

    stage(NFULL - 3, 0, prefetch2=NFULL - 1)
    stage(NFULL - 2, 1, prefetch2=None)
    stage(NFULL - 1, 0, prefetch2=None)
    wait_scatter(1)
    wait_scatter(0)

    toff = pl.multiple_of(ebase + NFULL * CE, 8)
    pltpu.sync_copy(dst_h.at[pl.ds(toff, TAIL_E)], didx_t)
    pltpu.sync_copy(msg2.at[pl.ds(toff, TAIL_E)],
                    m_buf.at[0, pl.ds(0, TAIL_E)])
    pltpu.sync_copy(m_buf.at[0, pl.ds(0, TAIL_E)], shared.at[didx_t],
                    add=True)

    plsc.subcore_barrier()

    @pl.when(sid < NS - 1)
    def _pub_a():
        pltpu.sync_copy(shared.at[pl.ds(row0, ROWS_A)],
                        p_out.at[cid, pl.ds(row0, ROWS_A)])

    @pl.when(sid == NS - 1)
    def _pub_b():
        pltpu.sync_copy(shared.at[pl.ds(row0, ROWS_LAST)],
                        p_out.at[cid, pl.ds(row0, ROWS_LAST)])


def _make_msg_call():
    mesh = plsc.VectorSubcoreMesh(core_axis_name="c", subcore_axis_name="s")
    return pl.kernel(
        _msg_body,
        out_type=[
            jax.ShapeDtypeStruct((E * F,), jnp.float32),
            jax.ShapeDtypeStruct((NPAD * F,), jnp.float32),
        ],
        mesh=mesh,
        compiler_params=pltpu.CompilerParams(needs_layout_passes=False),
        scratch_types=[
            pltpu.VMEM((2, F, F, CE), jnp.float32),
            pltpu.VMEM((2, CE, 128), jnp.float32),
            pltpu.VMEM((2, F, CE), jnp.float32),
            pltpu.VMEM((CE * F,), jnp.float32),
            pltpu.VMEM((CE * F,), jnp.float32),
            pltpu.VMEM((CE,), jnp.int32),
            pltpu.VMEM((CE,), jnp.int32),
            pltpu.SemaphoreType.DMA, pltpu.SemaphoreType.DMA,
            pltpu.SemaphoreType.DMA, pltpu.SemaphoreType.DMA,
            pltpu.SemaphoreType.DMA, pltpu.SemaphoreType.DMA,
            pltpu.SemaphoreType.DMA, pltpu.SemaphoreType.DMA,
        ],
    )


def _make_scatter_call():
    mesh = plsc.VectorSubcoreMesh(core_axis_name="c", subcore_axis_name="s")
    return pl.kernel(
        _scatter_body,
        out_type=jax.ShapeDtypeStruct((NC, N, F), jnp.float32),
        mesh=mesh,
        compiler_params=pltpu.CompilerParams(use_tc_tiling_on_sc=False),
        scratch_types=[
            pltpu.VMEM((2, CE, F), jnp.float32),
            pltpu.VMEM((2, CE, F), jnp.float32),
            pltpu.VMEM((2, CE), jnp.int32),
            pltpu.VMEM((2, CE), jnp.int32),
            pltpu.VMEM((TAIL_E,), jnp.int32),
            pltpu.VMEM_SHARED((N, F), jnp.float32),
            pltpu.SemaphoreType.DMA, pltpu.SemaphoreType.DMA,
            pltpu.SemaphoreType.DMA, pltpu.SemaphoreType.DMA,
        ],
    )


def kernel(feat, loop_weight, W, m_bias, h_bias, edge_index):
    src = edge_index[0]
    dst = edge_index[1]
    wt = jnp.transpose(W, (1, 2, 0))
    mbt = jnp.transpose(m_bias, (1, 2, 0)).reshape(F, E)
    lwt = jnp.pad(jnp.transpose(loop_weight, (1, 2, 0)),
                  ((0, 0), (0, 0), (0, NPAD - N)))
    hbt = jnp.pad(jnp.transpose(h_bias, (1, 2, 0)).reshape(F, N),
                  ((0, 0), (0, NPAD - N)))
    featp = jnp.pad(feat, ((0, NPAD - N), (0, 128 - F)))
    msg_f, loop_f = _make_msg_call()(wt, mbt, lwt, hbt, featp, src)
    p = _make_scatter_call()(msg_f.reshape(E, F), dst)
    return p[0] + p[1] + loop_f.reshape(NPAD, F)[:N]


# speedup vs baseline: 7.3154x; 1.0552x over previous
import jax
import jax.numpy as jnp
from jax import lax
from jax.experimental import pallas as pl
from jax.experimental.pallas import tpu as pltpu
from jax.experimental.pallas import tpu_sc as plsc

N = 10000
E = 160000
F = 16

NC = 2
NS = 16
NW = NC * NS

CE = 128
NCH_E = E // CE
CPT = NCH_E // NW
NPAD = 10240
NCH_N = NPAD // CE

EPT = E // NW
NFULL = EPT // CE
TAIL_E = EPT - NFULL * CE

ROWS_A = 624
ROWS_LAST = N - (NS - 1) * ROWS_A

_GDN = lax.GatherDimensionNumbers(
    offset_dims=(), collapsed_slice_dims=(0,), start_index_map=(0,))


def _gather16(x, idx):
    return lax.gather(x, idx.reshape(F, 1), dimension_numbers=_GDN,
                      slice_sizes=(1,),
                      mode=lax.GatherScatterMode.PROMISE_IN_BOUNDS)


def _iota16():
    return lax.iota(jnp.int32, F)


def _transpose16(v):
    iota = _iota16()
    for s in range(4):
        d = 1 << s
        idx = iota ^ d
        mask = (iota & d) == 0
        nv = list(v)
        for j in range(F):
            if j & d == 0:
                a, b = v[j], v[j | d]
                nv[j] = jnp.where(mask, a, _gather16(b, idx))
                nv[j | d] = jnp.where(mask, _gather16(a, idx), b)
        v = nv
    return v


def _compute_chunk_t(x_ref, w_ref, b_ref, m_ref):
    iota16 = _iota16() * F

    def blk(b_i):
        xv = [x_ref[b_i * F + j, pl.ds(0, F)] for j in range(F)]
        xt = _transpose16(xv)
        col = pl.ds(pl.multiple_of(b_i * F, 16), F)
        base = b_i * (F * F)
        for o in range(F):
            a0 = b_ref[o, col]
            a1 = xt[1] * w_ref[1, o, col]
            a2 = xt[2] * w_ref[2, o, col]
            a3 = xt[3] * w_ref[3, o, col]
            a0 = a0 + xt[0] * w_ref[0, o, col]
            for i in range(4, F, 4):
                a0 = a0 + xt[i] * w_ref[i, o, col]
                a1 = a1 + xt[i + 1] * w_ref[i + 1, o, col]
                a2 = a2 + xt[i + 2] * w_ref[i + 2, o, col]
                a3 = a3 + xt[i + 3] * w_ref[i + 3, o, col]
            acc = (a0 + a1) + (a2 + a3)
            idxv = iota16 + (base + o)
            plsc.store_scatter(m_ref, [idxv], acc)
    plsc.parallel_loop(0, CE // F, 1)(blk)


def _msg_body(wt, mbt, lwt, hbt, featp, ei, msg_f, loop_f,
              w_buf, x_buf, b_buf, m0, m1, sidx0, sidx1,
              si0, si1, sa0, sa1, sg0, sg1, so0, so1):
    cid = lax.axis_index("c")
    sid = lax.axis_index("s")
    wid = cid * NS + sid
    si = (si0, si1)
    sa = (sa0, sa1)
    sg = (sg0, sg1)
    so = (so0, so1)
    m1d = (m0, m1)
    sidx = (sidx0, sidx1)

    cb = wid * CPT

    def c128(ch):
        return pl.multiple_of(ch * CE, 128)

    def issue_idx(ch, p):
        pltpu.async_copy(ei.at[0, pl.ds(c128(ch), CE)], sidx[p], si[p])

    def wait_idx(p):
        pltpu.make_async_copy(ei.at[0, pl.ds(0, CE)], sidx[p], si[p]).wait()

    def issue_gather(p):
        pltpu.async_copy(featp.at[sidx[p]], x_buf.at[p], sg[p])

    def wait_gather(p):
        pltpu.make_async_copy(featp.at[pl.ds(0, CE)], x_buf.at[p],
                              sg[p]).wait()

    def issue_bulk(ch, p, w_src, b_src):
        pltpu.async_copy(w_src.at[:, :, pl.ds(c128(ch), CE)],
                         w_buf.at[p], sa[p])
        pltpu.async_copy(b_src.at[:, pl.ds(c128(ch), CE)],
                         b_buf.at[p], sa[p])

    def wait_bulk(p):
        pltpu.make_async_copy(wt.at[:, :, pl.ds(0, CE)], w_buf.at[p],
                              sa[p]).wait()
        pltpu.make_async_copy(mbt.at[:, pl.ds(0, CE)], b_buf.at[p],
                              sa[p]).wait()

    def issue_out(ch, p):
        pltpu.async_copy(
            m1d[p],
            msg_f.at[pl.ds(pl.multiple_of(ch * (CE * F), 8), CE * F)],
            so[p])

    def wait_out(p):
        pltpu.make_async_copy(msg_f.at[pl.ds(0, CE * F)], m1d[p],
                              so[p]).wait()

    def stage_g(c, p):
        q = 1 - p
        wait_idx(q)
        issue_gather(q)
        issue_bulk(cb + c + 1, q, wt, mbt)

        @pl.when(c >= 2)
        def _drain_out():
            wait_out(p)
        wait_gather(p)
        wait_bulk(p)

        @pl.when(c + 2 <= CPT - 1)
        def _prefetch():
            issue_idx(cb + c + 2, p)
        _compute_chunk_t(x_buf.at[p], w_buf.at[p], b_buf.at[p], m1d[p])
        issue_out(cb + c, p)

    issue_idx(cb + 0, 0)
    issue_idx(cb + 1, 1)
    wait_idx(0)
    issue_gather(0)
    issue_bulk(cb, 0, wt, mbt)

    def pair_body(t, _):
        ce = 2 * t
        stage_g(ce, 0)
        stage_g(ce + 1, 1)
        return 0
    lax.fori_loop(0, (CPT - 1) // 2, pair_body, 0)

    wait_out(0)
    wait_gather(0)
    wait_bulk(0)
    _compute_chunk_t(x_buf.at[0], w_buf.at[0], b_buf.at[0], m1d[0])
    issue_out(cb + CPT - 1, 0)
    wait_out(1)
    wait_out(0)

    def sync_edge_chunk(ch):
        pltpu.sync_copy(ei.at[0, pl.ds(c128(ch), CE)], sidx[0])
        d1 = pltpu.async_copy(featp.at[sidx[0]], x_buf.at[0], sg[0])
        d2 = pltpu.async_copy(wt.at[:, :, pl.ds(c128(ch), CE)],
                              w_buf.at[0], sa[0])
        d3 = pltpu.async_copy(mbt.at[:, pl.ds(c128(ch), CE)],
                              b_buf.at[0], sa[0])
        d1.wait(); d2.wait(); d3.wait()
        _compute_chunk_t(x_buf.at[0], w_buf.at[0], b_buf.at[0], m1d[0])
        pltpu.sync_copy(
            m1d[0],
            msg_f.at[pl.ds(pl.multiple_of(ch * (CE * F), 8), CE * F)])

    @pl.when(wid < NCH_E - NW * CPT)
    def _leftover_edges():
        sync_edge_chunk(NW * CPT + wid)

    def sync_node_chunk(ch):
        d0 = pltpu.async_copy(featp.at[pl.ds(c128(ch), CE)],
                              x_buf.at[0], sg[0])
        d2 = pltpu.async_copy(lwt.at[:, :, pl.ds(c128(ch), CE)],
                              w_buf.at[0], sa[0])
        d3 = pltpu.async_copy(hbt.at[:, pl.ds(c128(ch), CE)],
                              b_buf.at[0], sa[0])
        d0.wait(); d2.wait(); d3.wait()
        _compute_chunk_t(x_buf.at[0], w_buf.at[0], b_buf.at[0], m1d[0])
        pltpu.sync_copy(
            m1d[0],
            loop_f.at[pl.ds(pl.multiple_of(ch * (CE * F), 8), CE * F)])

    node_cnt = jnp.where(wid < NCH_N - 2 * NW, 3, 2)

    def node_body(k, _):
        ch = jnp.where(k < 2, 2 * wid + k, 2 * NW + wid)
        sync_node_chunk(ch)
        return 0
    lax.fori_loop(0, node_cnt, node_body, 0)


def _scatter_body(msg2, loop2, dst_h, p_out,
                  m_buf, m_s, didx, didx_s, didx_t, shared,
                  si0, si1, ss0, ss1):
    cid = lax.axis_index("c")
    sid = lax.axis_index("s")
    wid = cid * NS + sid
    si = (si0, si1)
    ss = (ss0, ss1)

    row0 = pl.multiple_of(sid * ROWS_A, 8)

    @pl.when(cid == 0)
    def _init_loop():
        @pl.when(sid < NS - 1)
        def _a():
            pltpu.sync_copy(loop2.at[pl.ds(row0, ROWS_A)],
                            shared.at[pl.ds(row0, ROWS_A)])

        @pl.when(sid == NS - 1)
        def _b():
            pltpu.sync_copy(loop2.at[pl.ds(row0, ROWS_LAST)],
                            shared.at[pl.ds(row0, ROWS_LAST)])

    @pl.when(cid == 1)
    def _init_zero():
        def zbody(i, _):
            m_buf[0, i, :] = jnp.zeros((F,), jnp.float32)
            return 0
        lax.fori_loop(0, CE, zbody, 0)
        for k in range(4):
            pltpu.sync_copy(
                m_buf.at[0],
                shared.at[pl.ds(pl.multiple_of(row0 + k * CE, 8), CE)])

        @pl.when(sid < NS - 1)
        def _zero_tail_a():
            pltpu.sync_copy(m_buf.at[0, pl.ds(0, ROWS_A - 4 * CE)],
                            shared.at[pl.ds(pl.multiple_of(row0 + 4 * CE, 8),
                                            ROWS_A - 4 * CE)])

        @pl.when(sid == NS - 1)
        def _zero_tail_b():
            pltpu.sync_copy(
                m_buf.at[0],
                shared.at[pl.ds(pl.multiple_of(row0 + 4 * CE, 8), CE)])

    plsc.subcore_barrier()

    ebase = wid * EPT

    def eoff(c):
        return pl.multiple_of(ebase + c * CE, 8)

    def issue_in(c, p):
        pltpu.async_copy(dst_h.at[pl.ds(eoff(c), CE)], didx.at[p], si[p])
        pltpu.async_copy(msg2.at[pl.ds(eoff(c), CE)], m_buf.at[p], si[p])

    def wait_in(p):
        pltpu.make_async_copy(dst_h.at[pl.ds(0, CE)], didx.at[p],
                              si[p]).wait()
        pltpu.make_async_copy(msg2.at[pl.ds(0, CE)], m_buf.at[p],
                              si[p]).wait()

    def issue_scatter(p):
        pltpu.async_copy(m_s.at[p], shared.at[didx_s.at[p]], ss[p],
                         add=True)

    def wait_scatter(p):
        pltpu.make_async_copy(msg2.at[pl.ds(0, CE)], m_s.at[p],
                              ss[p]).wait()

    def save(p):
        for k in range(CE // F):
            didx_s[p, pl.ds(k * F, F)] = didx[p, pl.ds(k * F, F)]

        def cp(i, _):
            m_s[p, i, :] = m_buf[p, i, :]
            return 0
        lax.fori_loop(0, CE, cp, 0)

    def stage(c, p, wait_prev_scatter=True, prefetch2=None):
        if wait_prev_scatter:
            wait_scatter(p)
        wait_in(p)
        save(p)
        if prefetch2 is not None:
            issue_in(prefetch2, p)
        issue_scatter(p)

    issue_in(0, 0)
    issue_in(1, 1)
    stage(0, 0, wait_prev_scatter=False, prefetch2=2)
    stage(1, 1, wait_prev_scatter=False, prefetch2=3)

    def pair_body(t, _):
        ce = 2 + 2 * t
        stage(ce, 0, prefetch2=ce + 2)
        stage(ce + 1, 1, prefetch2=ce + 3)
        return 0
    lax.fori_loop(0, (NFULL - 5) // 2, pair_body, 0)

    stage(NFULL - 3, 0, prefetch2=NFULL - 1)
    stage(NFULL - 2, 1, prefetch2=None)
    stage(NFULL - 1, 0, prefetch2=None)
    wait_scatter(1)
    wait_scatter(0)

    toff = pl.multiple_of(ebase + NFULL * CE, 8)
    pltpu.sync_copy(dst_h.at[pl.ds(toff, TAIL_E)], didx_t)
    pltpu.sync_copy(msg2.at[pl.ds(toff, TAIL_E)],
                    m_buf.at[0, pl.ds(0, TAIL_E)])
    pltpu.sync_copy(m_buf.at[0, pl.ds(0, TAIL_E)], shared.at[didx_t],
                    add=True)

    plsc.subcore_barrier()

    @pl.when(sid < NS - 1)
    def _pub_a():
        pltpu.sync_copy(shared.at[pl.ds(row0, ROWS_A)],
                        p_out.at[cid, pl.ds(row0, ROWS_A)])

    @pl.when(sid == NS - 1)
    def _pub_b():
        pltpu.sync_copy(shared.at[pl.ds(row0, ROWS_LAST)],
                        p_out.at[cid, pl.ds(row0, ROWS_LAST)])


def _make_msg_call():
    mesh = plsc.VectorSubcoreMesh(core_axis_name="c", subcore_axis_name="s")
    return pl.kernel(
        _msg_body,
        out_type=[
            jax.ShapeDtypeStruct((E * F,), jnp.float32),
            jax.ShapeDtypeStruct((NPAD * F,), jnp.float32),
        ],
        mesh=mesh,
        compiler_params=pltpu.CompilerParams(needs_layout_passes=False),
        scratch_types=[
            pltpu.VMEM((2, F, F, CE), jnp.float32),
            pltpu.VMEM((2, CE, 128), jnp.float32),
            pltpu.VMEM((2, F, CE), jnp.float32),
            pltpu.VMEM((CE * F,), jnp.float32),
            pltpu.VMEM((CE * F,), jnp.float32),
            pltpu.VMEM((CE,), jnp.int32),
            pltpu.VMEM((CE,), jnp.int32),
            pltpu.SemaphoreType.DMA, pltpu.SemaphoreType.DMA,
            pltpu.SemaphoreType.DMA, pltpu.SemaphoreType.DMA,
            pltpu.SemaphoreType.DMA, pltpu.SemaphoreType.DMA,
            pltpu.SemaphoreType.DMA, pltpu.SemaphoreType.DMA,
        ],
    )


def _make_scatter_call():
    mesh = plsc.VectorSubcoreMesh(core_axis_name="c", subcore_axis_name="s")
    return pl.kernel(
        _scatter_body,
        out_type=jax.ShapeDtypeStruct((NC, N, F), jnp.float32),
        mesh=mesh,
        compiler_params=pltpu.CompilerParams(use_tc_tiling_on_sc=False),
        scratch_types=[
            pltpu.VMEM((2, CE, F), jnp.float32),
            pltpu.VMEM((2, CE, F), jnp.float32),
            pltpu.VMEM((2, CE), jnp.int32),
            pltpu.VMEM((2, CE), jnp.int32),
            pltpu.VMEM((TAIL_E,), jnp.int32),
            pltpu.VMEM_SHARED((N, F), jnp.float32),
            pltpu.SemaphoreType.DMA, pltpu.SemaphoreType.DMA,
            pltpu.SemaphoreType.DMA, pltpu.SemaphoreType.DMA,
        ],
    )


def kernel(feat, loop_weight, W, m_bias, h_bias, edge_index):
    dst = edge_index[1]
    wt = jnp.transpose(W, (1, 2, 0))
    mbt = jnp.transpose(m_bias, (1, 2, 0)).reshape(F, E)
    lwt = jnp.pad(jnp.transpose(loop_weight, (1, 2, 0)),
                  ((0, 0), (0, 0), (0, NPAD - N)))
    hbt = jnp.pad(jnp.transpose(h_bias, (1, 2, 0)).reshape(F, N),
                  ((0, 0), (0, NPAD - N)))
    featp = jnp.pad(feat, ((0, NPAD - N), (0, 128 - F)))
    msg_f, loop_f = _make_msg_call()(wt, mbt, lwt, hbt, featp, edge_index)
    p = _make_scatter_call()(msg_f.reshape(E, F),
                             loop_f.reshape(NPAD, F), dst)
    return p[0] + p[1]
